# Initial kernel scaffold; baseline (speedup 1.0000x reference)
import jax
import jax.numpy as jnp
from jax.experimental import pallas as pl


def kernel(bx_gt):
    raise NotImplementedError("write your pallas kernel here")


# speedup vs baseline: 4.6795x; 4.6795x over previous
import numpy as np
import jax
import jax.numpy as jnp
from jax.experimental import pallas as pl
from jax.experimental.pallas import tpu as pltpu

_SIZE_IMG = 512
_STRIDE = 32
_N_ANCHOR = 9
_EPS = 1e-4
_IOU_SCALE = 10000.0
_NEG_TH_ACGT = 3000.0
_POS_TH_ACGT = 5000.0
_NEG_TH_GTAC = 100.0
_B = 64
_N_GT = 128


def _anchor_constants():
    hf = _SIZE_IMG // _STRIDE
    wf = _SIZE_IMG // _STRIDE
    smax = 2 ** _SIZE_IMG.bit_length()
    scales = np.array([smax >> 3, smax >> 2, smax >> 1], dtype=np.float32)
    sqrt2 = 1.4142135624
    ratios = np.array([[sqrt2, sqrt2 / 2.0], [1.0, 1.0], [sqrt2 / 2.0, sqrt2]],
                      dtype=np.float32)
    hw_one = np.concatenate([np.outer(scales, ratios[i]) for i in range(3)], axis=0)
    vy = np.arange(hf, dtype=np.float32)
    vx = np.arange(wf, dtype=np.float32)
    yy, xx = np.meshgrid(vy, vx, indexing='ij')
    coords = np.stack([yy, xx], axis=-1)[:, :, None, :] * _STRIDE + _STRIDE // 2
    coords = np.tile(coords, (1, 1, _N_ANCHOR, 1))
    hw = np.tile(hw_one[None, None, :, :], (hf, wf, 1, 1))
    ac_abs = np.concatenate([coords - 0.5 * hw, coords + 0.5 * hw], axis=-1).reshape(-1, 4)
    ac = (ac_abs / float(_SIZE_IMG)).astype(np.float32)
    mask = ((ac[:, 0] >= -0.2) & (ac[:, 1] >= -0.2)
            & (ac[:, 2] <= 1.2) & (ac[:, 3] <= 1.2)
            & (ac[:, 2] > ac[:, 0]) & (ac[:, 3] > ac[:, 1]))
    ac = ac[mask]
    h_r = np.maximum(ac[:, 2] - ac[:, 0], np.float32(_EPS))
    w_r = np.maximum(ac[:, 3] - ac[:, 1], np.float32(_EPS))
    yctr = ac[:, 0] + np.float32(0.5) * h_r
    xctr = ac[:, 1] + np.float32(0.5) * w_r
    area = (ac[:, 2] - ac[:, 0]) * (ac[:, 3] - ac[:, 1])
    cols = np.stack([ac[:, 0], ac[:, 1], ac[:, 2], ac[:, 3],
                     area, h_r, w_r, yctr, xctr], axis=1).astype(np.float32)
    return cols


_AC_CONST = _anchor_constants()
_N_AC = _AC_CONST.shape[0]


def _rpn_body(gt_ref, ac_ref, delta_ref, label_ref):
    gy0 = gt_ref[0, 0:1, :]
    gx0 = gt_ref[0, 1:2, :]
    gy1 = gt_ref[0, 2:3, :]
    gx1 = gt_ref[0, 3:4, :]
    ay0 = ac_ref[:, 0:1]
    ax0 = ac_ref[:, 1:2]
    ay1 = ac_ref[:, 2:3]
    ax1 = ac_ref[:, 3:4]
    area_ac = ac_ref[:, 4:5]

    iy = jnp.maximum(jnp.minimum(ay1, gy1) - jnp.maximum(ay0, gy0), 0.0)
    ix = jnp.maximum(jnp.minimum(ax1, gx1) - jnp.maximum(ax0, gx0), 0.0)
    inter = iy * ix
    area_gt = (gy1 - gy0) * (gx1 - gx0)
    union = area_ac + area_gt - inter
    iou = inter / (union + _EPS)
    s = jnp.round(iou * _IOU_SCALE)

    m = jnp.max(s, axis=1, keepdims=True)
    gmax = jnp.max(s, axis=0, keepdims=True)
    lane = jax.lax.broadcasted_iota(jnp.int32, (_N_AC, _N_GT), 1)
    bg = jnp.min(jnp.where(s == m, lane, _N_GT), axis=1, keepdims=True)

    isb = (s == gmax) & (s > _NEG_TH_GTAC)
    posx = jnp.any(isb, axis=1, keepdims=True)
    pos = (m >= _POS_TH_ACGT) | posx
    neg = (m < _NEG_TH_ACGT) & jnp.logical_not(pos)
    label_ref[0] = jnp.where(pos, 1.0, jnp.where(neg, 0.0, -1.0))

    onehot = lane == bg
    my0 = jnp.sum(jnp.where(onehot, jnp.broadcast_to(gy0, (_N_AC, _N_GT)), 0.0),
                  axis=1, keepdims=True)
    mx0 = jnp.sum(jnp.where(onehot, jnp.broadcast_to(gx0, (_N_AC, _N_GT)), 0.0),
                  axis=1, keepdims=True)
    my1 = jnp.sum(jnp.where(onehot, jnp.broadcast_to(gy1, (_N_AC, _N_GT)), 0.0),
                  axis=1, keepdims=True)
    mx1 = jnp.sum(jnp.where(onehot, jnp.broadcast_to(gx1, (_N_AC, _N_GT)), 0.0),
                  axis=1, keepdims=True)

    h_r = ac_ref[:, 5:6]
    w_r = ac_ref[:, 6:7]
    yctr_r = ac_ref[:, 7:8]
    xctr_r = ac_ref[:, 8:9]
    h_l = my1 - my0
    w_l = mx1 - mx0
    yctr_l = my0 + 0.5 * h_l
    xctr_l = mx0 + 0.5 * w_l
    delta_ref[0, :, 0:1] = (xctr_l - xctr_r) / w_r
    delta_ref[0, :, 1:2] = (yctr_l - yctr_r) / h_r
    delta_ref[0, :, 2:3] = jnp.log(jnp.maximum(w_l, _EPS) / w_r)
    delta_ref[0, :, 3:4] = jnp.log(jnp.maximum(h_l, _EPS) / h_r)


def _rpn_call(bx_gt_t, ac_const, interpret=False):
    return pl.pallas_call(
        _rpn_body,
        grid=(_B,),
        in_specs=[
            pl.BlockSpec((1, 4, _N_GT), lambda b: (b, 0, 0)),
            pl.BlockSpec((_N_AC, 9), lambda b: (0, 0)),
        ],
        out_specs=[
            pl.BlockSpec((1, _N_AC, 4), lambda b: (b, 0, 0)),
            pl.BlockSpec((1, _N_AC, 1), lambda b: (b, 0, 0)),
        ],
        out_shape=[
            jax.ShapeDtypeStruct((_B, _N_AC, 4), jnp.float32),
            jax.ShapeDtypeStruct((_B, _N_AC, 1), jnp.float32),
        ],
        compiler_params=pltpu.CompilerParams(
            dimension_semantics=("arbitrary",),
        ),
        interpret=interpret,
    )(bx_gt_t, ac_const)


def kernel(bx_gt):
    bx_gt_t = jnp.transpose(bx_gt, (0, 2, 1))
    ac_const = jnp.asarray(_AC_CONST)
    delta, label = _rpn_call(bx_gt_t, ac_const)
    return delta, label.reshape(_B, _N_AC)


# speedup vs baseline: 14.5698x; 3.1136x over previous
import numpy as np
import jax
import jax.numpy as jnp
from jax.experimental import pallas as pl
from jax.experimental.pallas import tpu as pltpu

_SIZE_IMG = 512
_STRIDE = 32
_N_ANCHOR = 9
_EPS = 1e-4
_B = 64
_N_GT = 128


def _anchor_constants():
    hf = _SIZE_IMG // _STRIDE
    wf = _SIZE_IMG // _STRIDE
    smax = 2 ** _SIZE_IMG.bit_length()
    scales = np.array([smax >> 3, smax >> 2, smax >> 1], dtype=np.float32)
    sqrt2 = 1.4142135624
    ratios = np.array([[sqrt2, sqrt2 / 2.0], [1.0, 1.0], [sqrt2 / 2.0, sqrt2]],
                      dtype=np.float32)
    hw_one = np.concatenate([np.outer(scales, ratios[i]) for i in range(3)], axis=0)
    vy = np.arange(hf, dtype=np.float32)
    vx = np.arange(wf, dtype=np.float32)
    yy, xx = np.meshgrid(vy, vx, indexing='ij')
    coords = np.stack([yy, xx], axis=-1)[:, :, None, :] * _STRIDE + _STRIDE // 2
    coords = np.tile(coords, (1, 1, _N_ANCHOR, 1))
    hw = np.tile(hw_one[None, None, :, :], (hf, wf, 1, 1))
    ac_abs = np.concatenate([coords - 0.5 * hw, coords + 0.5 * hw], axis=-1).reshape(-1, 4)
    ac = (ac_abs / float(_SIZE_IMG)).astype(np.float32)
    mask = ((ac[:, 0] >= -0.2) & (ac[:, 1] >= -0.2)
            & (ac[:, 2] <= 1.2) & (ac[:, 3] <= 1.2)
            & (ac[:, 2] > ac[:, 0]) & (ac[:, 3] > ac[:, 1]))
    ac = ac[mask]
    h_r = np.maximum(ac[:, 2] - ac[:, 0], np.float32(_EPS))
    w_r = np.maximum(ac[:, 3] - ac[:, 1], np.float32(_EPS))
    yctr = ac[:, 0] + np.float32(0.5) * h_r
    xctr = ac[:, 1] + np.float32(0.5) * w_r
    area = (ac[:, 2] - ac[:, 0]) * (ac[:, 3] - ac[:, 1])
    rows = np.stack([ac[:, 0], ac[:, 1], ac[:, 2], ac[:, 3],
                     area, h_r, w_r, yctr, xctr], axis=0).astype(np.float32)
    return rows


_AC_CONST = _anchor_constants()
_N_AC = _AC_CONST.shape[1]


def _rpn_body(gt_ref, gtt_ref, ac_ref, delta_ref, label_ref):
    gy0 = gt_ref[0, :, 0:1]
    gx0 = gt_ref[0, :, 1:2]
    gy1 = gt_ref[0, :, 2:3]
    gx1 = gt_ref[0, :, 3:4]
    ay0 = ac_ref[0:1, :]
    ax0 = ac_ref[1:2, :]
    ay1 = ac_ref[2:3, :]
    ax1 = ac_ref[3:4, :]
    area_ac = ac_ref[4:5, :]

    iy = jnp.maximum(jnp.minimum(ay1, gy1) - jnp.maximum(ay0, gy0), 0.0)
    ix = jnp.maximum(jnp.minimum(ax1, gx1) - jnp.maximum(ax0, gx0), 0.0)
    inter = iy * ix
    area_gt = (gy1 - gy0) * (gx1 - gx0)
    union = area_ac + area_gt - inter
    iou = inter / (union + _EPS)
    s = jnp.round(iou * 10000.0).astype(jnp.int32)

    gt_idx = jax.lax.broadcasted_iota(jnp.int32, (_N_GT, _N_AC), 0)
    enc = s * 128 + (127 - gt_idx)
    enc_max = jnp.max(enc, axis=0, keepdims=True)
    m = enc_max >> 7
    bg = 127 - (enc_max & 127)

    gmax = jnp.max(s, axis=1, keepdims=True)
    isb = (s == gmax) & (s > 100)
    posx = jnp.any(isb, axis=0, keepdims=True)
    pos = (m >= 5000) | posx
    neg = (m < 3000) & jnp.logical_not(pos)
    label_ref[0] = jnp.where(pos, 1.0, jnp.where(neg, 0.0, -1.0))

    onehot = jnp.where(gt_idx == bg, 1.0, 0.0)
    gt4 = gtt_ref[0]
    matched = jax.lax.dot_general(
        gt4, onehot, (((1,), (0,)), ((), ())),
        precision=jax.lax.Precision.HIGHEST,
        preferred_element_type=jnp.float32)
    my0 = matched[0:1, :]
    mx0 = matched[1:2, :]
    my1 = matched[2:3, :]
    mx1 = matched[3:4, :]

    h_r = ac_ref[5:6, :]
    w_r = ac_ref[6:7, :]
    yctr_r = ac_ref[7:8, :]
    xctr_r = ac_ref[8:9, :]
    h_l = my1 - my0
    w_l = mx1 - mx0
    yctr_l = my0 + 0.5 * h_l
    xctr_l = mx0 + 0.5 * w_l
    delta_ref[0, 0:1, :] = (xctr_l - xctr_r) / w_r
    delta_ref[0, 1:2, :] = (yctr_l - yctr_r) / h_r
    delta_ref[0, 2:3, :] = jnp.log(jnp.maximum(w_l, _EPS) / w_r)
    delta_ref[0, 3:4, :] = jnp.log(jnp.maximum(h_l, _EPS) / h_r)


def _rpn_call(bx_gt, bx_gt_t, ac_const, interpret=False):
    return pl.pallas_call(
        _rpn_body,
        grid=(_B,),
        in_specs=[
            pl.BlockSpec((1, _N_GT, 4), lambda b: (b, 0, 0)),
            pl.BlockSpec((1, 4, _N_GT), lambda b: (b, 0, 0)),
            pl.BlockSpec((9, _N_AC), lambda b: (0, 0)),
        ],
        out_specs=[
            pl.BlockSpec((1, 4, _N_AC), lambda b: (b, 0, 0)),
            pl.BlockSpec((1, 1, _N_AC), lambda b: (b, 0, 0)),
        ],
        out_shape=[
            jax.ShapeDtypeStruct((_B, 4, _N_AC), jnp.float32),
            jax.ShapeDtypeStruct((_B, 1, _N_AC), jnp.float32),
        ],
        compiler_params=pltpu.CompilerParams(
            dimension_semantics=("arbitrary",),
        ),
        interpret=interpret,
    )(bx_gt, bx_gt_t, ac_const)


def kernel(bx_gt):
    bx_gt_t = jnp.transpose(bx_gt, (0, 2, 1))
    ac_const = jnp.asarray(_AC_CONST)
    delta_t, label = _rpn_call(bx_gt, bx_gt_t, ac_const)
    delta = jnp.transpose(delta_t, (0, 2, 1))
    return delta, label.reshape(_B, _N_AC)


# speedup vs baseline: 18.7558x; 1.2873x over previous
import numpy as np
import jax
import jax.numpy as jnp
from jax.experimental import pallas as pl
from jax.experimental.pallas import tpu as pltpu

_SIZE_IMG = 512
_STRIDE = 32
_N_ANCHOR = 9
_EPS = 1e-4
_B = 64
_N_GT = 128


def _anchor_constants():
    hf = _SIZE_IMG // _STRIDE
    wf = _SIZE_IMG // _STRIDE
    smax = 2 ** _SIZE_IMG.bit_length()
    scales = np.array([smax >> 3, smax >> 2, smax >> 1], dtype=np.float32)
    sqrt2 = 1.4142135624
    ratios = np.array([[sqrt2, sqrt2 / 2.0], [1.0, 1.0], [sqrt2 / 2.0, sqrt2]],
                      dtype=np.float32)
    hw_one = np.concatenate([np.outer(scales, ratios[i]) for i in range(3)], axis=0)
    vy = np.arange(hf, dtype=np.float32)
    vx = np.arange(wf, dtype=np.float32)
    yy, xx = np.meshgrid(vy, vx, indexing='ij')
    coords = np.stack([yy, xx], axis=-1)[:, :, None, :] * _STRIDE + _STRIDE // 2
    coords = np.tile(coords, (1, 1, _N_ANCHOR, 1))
    hw = np.tile(hw_one[None, None, :, :], (hf, wf, 1, 1))
    ac_abs = np.concatenate([coords - 0.5 * hw, coords + 0.5 * hw], axis=-1).reshape(-1, 4)
    ac = (ac_abs / float(_SIZE_IMG)).astype(np.float32)
    mask = ((ac[:, 0] >= -0.2) & (ac[:, 1] >= -0.2)
            & (ac[:, 2] <= 1.2) & (ac[:, 3] <= 1.2)
            & (ac[:, 2] > ac[:, 0]) & (ac[:, 3] > ac[:, 1]))
    ac = ac[mask]
    h_r = np.maximum(ac[:, 2] - ac[:, 0], np.float32(_EPS))
    w_r = np.maximum(ac[:, 3] - ac[:, 1], np.float32(_EPS))
    yctr = ac[:, 0] + np.float32(0.5) * h_r
    xctr = ac[:, 1] + np.float32(0.5) * w_r
    area = (ac[:, 2] - ac[:, 0]) * (ac[:, 3] - ac[:, 1])
    rows = np.stack([ac[:, 0], ac[:, 1], ac[:, 2], ac[:, 3],
                     area, h_r, w_r, yctr, xctr], axis=0).astype(np.float32)
    return rows


_AC_CONST = _anchor_constants()
_N_AC = _AC_CONST.shape[1]
_REV_CONST = np.broadcast_to(
    (127.0 - np.arange(_N_GT, dtype=np.float32))[:, None], (_N_GT, _N_AC)).copy()


def _rpn_body(gt_ref, gtt_ref, ac_ref, rev_ref, delta_ref, label_ref):
    gy0 = gt_ref[0, :, 0:1]
    gx0 = gt_ref[0, :, 1:2]
    gy1 = gt_ref[0, :, 2:3]
    gx1 = gt_ref[0, :, 3:4]
    ay0 = ac_ref[0:1, :]
    ax0 = ac_ref[1:2, :]
    ay1 = ac_ref[2:3, :]
    ax1 = ac_ref[3:4, :]
    area_ac = ac_ref[4:5, :]

    iy = jnp.maximum(jnp.minimum(ay1, gy1) - jnp.maximum(ay0, gy0), 0.0)
    ix = jnp.maximum(jnp.minimum(ax1, gx1) - jnp.maximum(ax0, gx0), 0.0)
    inter = iy * ix
    area_gt = (gy1 - gy0) * (gx1 - gx0)
    union = area_ac + area_gt - inter
    iou = inter / (union + _EPS)
    s = jnp.round(iou * 10000.0)

    enc = s * 128.0 + rev_ref[...]
    enc_max = jnp.max(enc, axis=0, keepdims=True)

    gmax = jnp.max(s, axis=1, keepdims=True)
    gmax2 = jnp.where(gmax > 100.0, gmax, -1.0)
    isb = s == gmax2
    posx = jnp.any(isb, axis=0, keepdims=True)
    pos = (enc_max >= 640000.0) | posx
    neg = (enc_max < 384000.0) & jnp.logical_not(pos)
    label_ref[0] = jnp.where(pos, 1.0, jnp.where(neg, 0.0, -1.0))

    onehot = jnp.where(enc == enc_max, 1.0, 0.0).astype(jnp.bfloat16)
    gt4 = gtt_ref[0]
    g1 = gt4.astype(jnp.bfloat16)
    r1 = gt4 - g1.astype(jnp.float32)
    g2 = r1.astype(jnp.bfloat16)
    g3 = (r1 - g2.astype(jnp.float32)).astype(jnp.bfloat16)
    dn = (((1,), (0,)), ((), ()))
    matched = (jax.lax.dot_general(g1, onehot, dn, preferred_element_type=jnp.float32)
               + jax.lax.dot_general(g2, onehot, dn, preferred_element_type=jnp.float32)
               + jax.lax.dot_general(g3, onehot, dn, preferred_element_type=jnp.float32))
    my0 = matched[0:1, :]
    mx0 = matched[1:2, :]
    my1 = matched[2:3, :]
    mx1 = matched[3:4, :]

    h_r = ac_ref[5:6, :]
    w_r = ac_ref[6:7, :]
    yctr_r = ac_ref[7:8, :]
    xctr_r = ac_ref[8:9, :]
    h_l = my1 - my0
    w_l = mx1 - mx0
    yctr_l = my0 + 0.5 * h_l
    xctr_l = mx0 + 0.5 * w_l
    delta_ref[0, 0:1, :] = (xctr_l - xctr_r) / w_r
    delta_ref[0, 1:2, :] = (yctr_l - yctr_r) / h_r
    delta_ref[0, 2:3, :] = jnp.log(jnp.maximum(w_l, _EPS) / w_r)
    delta_ref[0, 3:4, :] = jnp.log(jnp.maximum(h_l, _EPS) / h_r)


def _rpn_call(bx_gt, bx_gt_t, ac_const, rev_const, interpret=False):
    return pl.pallas_call(
        _rpn_body,
        grid=(_B,),
        in_specs=[
            pl.BlockSpec((1, _N_GT, 4), lambda b: (b, 0, 0)),
            pl.BlockSpec((1, 4, _N_GT), lambda b: (b, 0, 0)),
            pl.BlockSpec((9, _N_AC), lambda b: (0, 0)),
            pl.BlockSpec((_N_GT, _N_AC), lambda b: (0, 0)),
        ],
        out_specs=[
            pl.BlockSpec((1, 4, _N_AC), lambda b: (b, 0, 0)),
            pl.BlockSpec((1, 1, _N_AC), lambda b: (b, 0, 0)),
        ],
        out_shape=[
            jax.ShapeDtypeStruct((_B, 4, _N_AC), jnp.float32),
            jax.ShapeDtypeStruct((_B, 1, _N_AC), jnp.float32),
        ],
        compiler_params=pltpu.CompilerParams(
            dimension_semantics=("arbitrary",),
        ),
        interpret=interpret,
    )(bx_gt, bx_gt_t, ac_const, rev_const)


def kernel(bx_gt):
    bx_gt_t = jnp.transpose(bx_gt, (0, 2, 1))
    ac_const = jnp.asarray(_AC_CONST)
    rev_const = jnp.asarray(_REV_CONST)
    delta_t, label = _rpn_call(bx_gt, bx_gt_t, ac_const, rev_const)
    delta = jnp.transpose(delta_t, (0, 2, 1))
    return delta, label.reshape(_B, _N_AC)


# speedup vs baseline: 21.5742x; 1.1503x over previous
import numpy as np
import jax
import jax.numpy as jnp
from jax.experimental import pallas as pl
from jax.experimental.pallas import tpu as pltpu

_SIZE_IMG = 512
_STRIDE = 32
_N_ANCHOR = 9
_EPS = 1e-4
_B = 64
_N_GT = 128


def _anchor_constants():
    hf = _SIZE_IMG // _STRIDE
    wf = _SIZE_IMG // _STRIDE
    smax = 2 ** _SIZE_IMG.bit_length()
    scales = np.array([smax >> 3, smax >> 2, smax >> 1], dtype=np.float32)
    sqrt2 = 1.4142135624
    ratios = np.array([[sqrt2, sqrt2 / 2.0], [1.0, 1.0], [sqrt2 / 2.0, sqrt2]],
                      dtype=np.float32)
    hw_one = np.concatenate([np.outer(scales, ratios[i]) for i in range(3)], axis=0)
    vy = np.arange(hf, dtype=np.float32)
    vx = np.arange(wf, dtype=np.float32)
    yy, xx = np.meshgrid(vy, vx, indexing='ij')
    coords = np.stack([yy, xx], axis=-1)[:, :, None, :] * _STRIDE + _STRIDE // 2
    coords = np.tile(coords, (1, 1, _N_ANCHOR, 1))
    hw = np.tile(hw_one[None, None, :, :], (hf, wf, 1, 1))
    ac_abs = np.concatenate([coords - 0.5 * hw, coords + 0.5 * hw], axis=-1).reshape(-1, 4)
    ac = (ac_abs / float(_SIZE_IMG)).astype(np.float32)
    mask = ((ac[:, 0] >= -0.2) & (ac[:, 1] >= -0.2)
            & (ac[:, 2] <= 1.2) & (ac[:, 3] <= 1.2)
            & (ac[:, 2] > ac[:, 0]) & (ac[:, 3] > ac[:, 1]))
    ac = ac[mask]
    h_r = np.maximum(ac[:, 2] - ac[:, 0], np.float32(_EPS))
    w_r = np.maximum(ac[:, 3] - ac[:, 1], np.float32(_EPS))
    yctr = ac[:, 0] + np.float32(0.5) * h_r
    xctr = ac[:, 1] + np.float32(0.5) * w_r
    area = (ac[:, 2] - ac[:, 0]) * (ac[:, 3] - ac[:, 1])
    rows = np.stack([ac[:, 0], ac[:, 1], ac[:, 2], ac[:, 3],
                     area, h_r, w_r, yctr, xctr], axis=0).astype(np.float32)
    return rows


_AC_CONST = _anchor_constants()
_N_AC = _AC_CONST.shape[1]
_REV_CONST = np.broadcast_to(
    (127.0 - np.arange(_N_GT, dtype=np.float32))[:, None], (_N_GT, _N_AC)).copy()


_BB = 4


def _rpn_body(gt_ref, gtt_ref, ac_ref, rev_ref, delta_ref, label_ref):
    for k in range(_BB):
        _rpn_one(k, gt_ref, gtt_ref, ac_ref, rev_ref, delta_ref, label_ref)


def _rpn_one(k, gt_ref, gtt_ref, ac_ref, rev_ref, delta_ref, label_ref):
    gy0 = gt_ref[k, :, 0:1]
    gx0 = gt_ref[k, :, 1:2]
    gy1 = gt_ref[k, :, 2:3]
    gx1 = gt_ref[k, :, 3:4]
    ay0 = ac_ref[0:1, :]
    ax0 = ac_ref[1:2, :]
    ay1 = ac_ref[2:3, :]
    ax1 = ac_ref[3:4, :]
    area_ac = ac_ref[4:5, :]

    iy = jnp.maximum(jnp.minimum(ay1, gy1) - jnp.maximum(ay0, gy0), 0.0)
    ix = jnp.maximum(jnp.minimum(ax1, gx1) - jnp.maximum(ax0, gx0), 0.0)
    inter = iy * ix
    area_gt = (gy1 - gy0) * (gx1 - gx0)
    union = area_ac + area_gt - inter
    iou = inter / (union + _EPS)
    s = jnp.round(iou * 10000.0)

    enc = s * 128.0 + rev_ref[...]
    enc_max = jnp.max(enc, axis=0, keepdims=True)

    gmax = jnp.max(s, axis=1, keepdims=True)
    gmax2 = jnp.where(gmax > 100.0, gmax, -1.0)
    isb = s == gmax2
    posx = jnp.any(isb, axis=0, keepdims=True)
    pos = (enc_max >= 640000.0) | posx
    neg = (enc_max < 384000.0) & jnp.logical_not(pos)
    label_ref[k] = jnp.where(pos, 1.0, jnp.where(neg, 0.0, -1.0))

    onehot = jnp.where(enc == enc_max, 1.0, 0.0).astype(jnp.bfloat16)
    gt4 = gtt_ref[k]
    g1 = gt4.astype(jnp.bfloat16)
    r1 = gt4 - g1.astype(jnp.float32)
    g2 = r1.astype(jnp.bfloat16)
    g3 = (r1 - g2.astype(jnp.float32)).astype(jnp.bfloat16)
    dn = (((1,), (0,)), ((), ()))
    matched = (jax.lax.dot_general(g1, onehot, dn, preferred_element_type=jnp.float32)
               + jax.lax.dot_general(g2, onehot, dn, preferred_element_type=jnp.float32)
               + jax.lax.dot_general(g3, onehot, dn, preferred_element_type=jnp.float32))
    my0 = matched[0:1, :]
    mx0 = matched[1:2, :]
    my1 = matched[2:3, :]
    mx1 = matched[3:4, :]

    h_r = ac_ref[5:6, :]
    w_r = ac_ref[6:7, :]
    yctr_r = ac_ref[7:8, :]
    xctr_r = ac_ref[8:9, :]
    h_l = my1 - my0
    w_l = mx1 - mx0
    yctr_l = my0 + 0.5 * h_l
    xctr_l = mx0 + 0.5 * w_l
    delta_ref[k, 0:1, :] = (xctr_l - xctr_r) / w_r
    delta_ref[k, 1:2, :] = (yctr_l - yctr_r) / h_r
    delta_ref[k, 2:3, :] = jnp.log(jnp.maximum(w_l, _EPS) / w_r)
    delta_ref[k, 3:4, :] = jnp.log(jnp.maximum(h_l, _EPS) / h_r)


def _rpn_call(bx_gt, bx_gt_t, ac_const, rev_const, interpret=False):
    return pl.pallas_call(
        _rpn_body,
        grid=(_B // _BB,),
        in_specs=[
            pl.BlockSpec((_BB, _N_GT, 4), lambda b: (b, 0, 0)),
            pl.BlockSpec((_BB, 4, _N_GT), lambda b: (b, 0, 0)),
            pl.BlockSpec((9, _N_AC), lambda b: (0, 0)),
            pl.BlockSpec((_N_GT, _N_AC), lambda b: (0, 0)),
        ],
        out_specs=[
            pl.BlockSpec((_BB, 4, _N_AC), lambda b: (b, 0, 0)),
            pl.BlockSpec((_BB, 1, _N_AC), lambda b: (b, 0, 0)),
        ],
        out_shape=[
            jax.ShapeDtypeStruct((_B, 4, _N_AC), jnp.float32),
            jax.ShapeDtypeStruct((_B, 1, _N_AC), jnp.float32),
        ],
        compiler_params=pltpu.CompilerParams(
            dimension_semantics=("arbitrary",),
        ),
        interpret=interpret,
    )(bx_gt, bx_gt_t, ac_const, rev_const)


def kernel(bx_gt):
    bx_gt_t = jnp.transpose(bx_gt, (0, 2, 1))
    ac_const = jnp.asarray(_AC_CONST)
    rev_const = jnp.asarray(_REV_CONST)
    delta_t, label = _rpn_call(bx_gt, bx_gt_t, ac_const, rev_const)
    delta = jnp.transpose(delta_t, (0, 2, 1))
    return delta, label.reshape(_B, _N_AC)


# speedup vs baseline: 22.0834x; 1.0236x over previous
import numpy as np
import jax
import jax.numpy as jnp
from jax.experimental import pallas as pl
from jax.experimental.pallas import tpu as pltpu

_SIZE_IMG = 512
_STRIDE = 32
_N_ANCHOR = 9
_EPS = 1e-4
_B = 64
_N_GT = 128


def _anchor_constants():
    hf = _SIZE_IMG // _STRIDE
    wf = _SIZE_IMG // _STRIDE
    smax = 2 ** _SIZE_IMG.bit_length()
    scales = np.array([smax >> 3, smax >> 2, smax >> 1], dtype=np.float32)
    sqrt2 = 1.4142135624
    ratios = np.array([[sqrt2, sqrt2 / 2.0], [1.0, 1.0], [sqrt2 / 2.0, sqrt2]],
                      dtype=np.float32)
    hw_one = np.concatenate([np.outer(scales, ratios[i]) for i in range(3)], axis=0)
    vy = np.arange(hf, dtype=np.float32)
    vx = np.arange(wf, dtype=np.float32)
    yy, xx = np.meshgrid(vy, vx, indexing='ij')
    coords = np.stack([yy, xx], axis=-1)[:, :, None, :] * _STRIDE + _STRIDE // 2
    coords = np.tile(coords, (1, 1, _N_ANCHOR, 1))
    hw = np.tile(hw_one[None, None, :, :], (hf, wf, 1, 1))
    ac_abs = np.concatenate([coords - 0.5 * hw, coords + 0.5 * hw], axis=-1).reshape(-1, 4)
    ac = (ac_abs / float(_SIZE_IMG)).astype(np.float32)
    mask = ((ac[:, 0] >= -0.2) & (ac[:, 1] >= -0.2)
            & (ac[:, 2] <= 1.2) & (ac[:, 3] <= 1.2)
            & (ac[:, 2] > ac[:, 0]) & (ac[:, 3] > ac[:, 1]))
    ac = ac[mask]
    h_r = np.maximum(ac[:, 2] - ac[:, 0], np.float32(_EPS))
    w_r = np.maximum(ac[:, 3] - ac[:, 1], np.float32(_EPS))
    yctr = ac[:, 0] + np.float32(0.5) * h_r
    xctr = ac[:, 1] + np.float32(0.5) * w_r
    area = (ac[:, 2] - ac[:, 0]) * (ac[:, 3] - ac[:, 1])
    rows = np.stack([ac[:, 0], ac[:, 1], ac[:, 2], ac[:, 3],
                     area, h_r, w_r, yctr, xctr], axis=0).astype(np.float32)
    return rows


_AC_CONST = _anchor_constants()
_N_AC = _AC_CONST.shape[1]
_REV_CONST = np.broadcast_to(
    (127.0 - np.arange(_N_GT, dtype=np.float32))[:, None], (_N_GT, _N_AC)).copy()


_BB = 8


def _rpn_body(gt_ref, gtt_ref, ac_ref, rev_ref, delta_ref, label_ref):
    for k in range(_BB):
        _rpn_one(k, gt_ref, gtt_ref, ac_ref, rev_ref, delta_ref, label_ref)


def _rpn_one(k, gt_ref, gtt_ref, ac_ref, rev_ref, delta_ref, label_ref):
    gy0 = gt_ref[k, :, 0:1]
    gx0 = gt_ref[k, :, 1:2]
    gy1 = gt_ref[k, :, 2:3]
    gx1 = gt_ref[k, :, 3:4]
    ay0 = ac_ref[0:1, :]
    ax0 = ac_ref[1:2, :]
    ay1 = ac_ref[2:3, :]
    ax1 = ac_ref[3:4, :]
    area_ac = ac_ref[4:5, :]

    iy = jnp.maximum(jnp.minimum(ay1, gy1) - jnp.maximum(ay0, gy0), 0.0)
    ix = jnp.maximum(jnp.minimum(ax1, gx1) - jnp.maximum(ax0, gx0), 0.0)
    inter = iy * ix
    area_gt = (gy1 - gy0) * (gx1 - gx0)
    union = area_ac + area_gt - inter
    iou = inter / (union + _EPS)
    s = jnp.round(iou * 10000.0)

    enc = s * 128.0 + rev_ref[...]
    enc_max = jnp.max(enc, axis=0, keepdims=True)

    gmax = jnp.max(s, axis=1, keepdims=True)
    gmax2 = jnp.where(gmax > 100.0, gmax, -1.0)
    isb = s == gmax2
    posx = jnp.any(isb, axis=0, keepdims=True)
    pos = (enc_max >= 640000.0) | posx
    neg = (enc_max < 384000.0) & jnp.logical_not(pos)
    label_ref[k] = jnp.where(pos, 1.0, jnp.where(neg, 0.0, -1.0))

    onehot = jnp.where(enc == enc_max, 1.0, 0.0).astype(jnp.bfloat16)
    gt4 = gtt_ref[k]
    g1 = gt4.astype(jnp.bfloat16)
    r1 = gt4 - g1.astype(jnp.float32)
    g2 = r1.astype(jnp.bfloat16)
    g3 = (r1 - g2.astype(jnp.float32)).astype(jnp.bfloat16)
    dn = (((1,), (0,)), ((), ()))
    matched = (jax.lax.dot_general(g1, onehot, dn, preferred_element_type=jnp.float32)
               + jax.lax.dot_general(g2, onehot, dn, preferred_element_type=jnp.float32)
               + jax.lax.dot_general(g3, onehot, dn, preferred_element_type=jnp.float32))
    my0 = matched[0:1, :]
    mx0 = matched[1:2, :]
    my1 = matched[2:3, :]
    mx1 = matched[3:4, :]

    h_r = ac_ref[5:6, :]
    w_r = ac_ref[6:7, :]
    yctr_r = ac_ref[7:8, :]
    xctr_r = ac_ref[8:9, :]
    h_l = my1 - my0
    w_l = mx1 - mx0
    yctr_l = my0 + 0.5 * h_l
    xctr_l = mx0 + 0.5 * w_l
    delta_ref[k, 0:1, :] = (xctr_l - xctr_r) / w_r
    delta_ref[k, 1:2, :] = (yctr_l - yctr_r) / h_r
    delta_ref[k, 2:3, :] = jnp.log(jnp.maximum(w_l, _EPS) / w_r)
    delta_ref[k, 3:4, :] = jnp.log(jnp.maximum(h_l, _EPS) / h_r)


def _rpn_call(bx_gt, bx_gt_t, ac_const, rev_const, interpret=False):
    return pl.pallas_call(
        _rpn_body,
        grid=(_B // _BB,),
        in_specs=[
            pl.BlockSpec((_BB, _N_GT, 4), lambda b: (b, 0, 0)),
            pl.BlockSpec((_BB, 4, _N_GT), lambda b: (b, 0, 0)),
            pl.BlockSpec((9, _N_AC), lambda b: (0, 0)),
            pl.BlockSpec((_N_GT, _N_AC), lambda b: (0, 0)),
        ],
        out_specs=[
            pl.BlockSpec((_BB, 4, _N_AC), lambda b: (b, 0, 0)),
            pl.BlockSpec((_BB, 1, _N_AC), lambda b: (b, 0, 0)),
        ],
        out_shape=[
            jax.ShapeDtypeStruct((_B, 4, _N_AC), jnp.float32),
            jax.ShapeDtypeStruct((_B, 1, _N_AC), jnp.float32),
        ],
        compiler_params=pltpu.CompilerParams(
            dimension_semantics=("arbitrary",),
        ),
        interpret=interpret,
    )(bx_gt, bx_gt_t, ac_const, rev_const)


def kernel(bx_gt):
    bx_gt_t = jnp.transpose(bx_gt, (0, 2, 1))
    ac_const = jnp.asarray(_AC_CONST)
    rev_const = jnp.asarray(_REV_CONST)
    delta_t, label = _rpn_call(bx_gt, bx_gt_t, ac_const, rev_const)
    delta = jnp.transpose(delta_t, (0, 2, 1))
    return delta, label.reshape(_B, _N_AC)
